# baseline (speedup 1.0000x reference)
import functools

import jax
import jax.numpy as jnp
from jax import lax
from jax.experimental import pallas as pl
from jax.experimental.pallas import tpu as pltpu
from jax.experimental.pallas import tpu_sc as plsc

_NUM_CLASSES = 80
_SCORE_THRESHOLD = 0.05
_IOU_THRESHOLD = 0.5
_MAX_DET = 100

_N = 20000
_B = 8
_ROWS = 160
_NPAD = _ROWS * 128
_BIG = 2 ** 30
_PADBOX = 1e30


_GDN = lax.GatherDimensionNumbers(offset_dims=(), collapsed_slice_dims=(0,),
                                  start_index_map=(0,))


def _shuffle(v, idx):
    return lax.gather(v, idx[:, None], _GDN, (1,),
                      mode=lax.GatherScatterMode.PROMISE_IN_BOUNDS)


def _vmax(v):
    iota = lax.iota(jnp.int32, 16)
    for sh in (8, 4, 2, 1):
        v = jnp.maximum(v, _shuffle(v, (iota + sh) % 16))
    return v[0]


def _vmini(v):
    iota = lax.iota(jnp.int32, 16)
    for sh in (8, 4, 2, 1):
        v = jnp.minimum(v, _shuffle(v, (iota + sh) % 16))
    return v[0]

def _score_label_body(cls_ref, sc_ref, lb_ref):
    x = jnp.transpose(cls_ref[0])
    m = jnp.max(x, axis=0)
    cls_iota = jax.lax.broadcasted_iota(jnp.int32, x.shape, 0)
    lab = jnp.min(jnp.where(x == m[None, :], cls_iota, _BIG),
                  axis=0).astype(jnp.float32)
    sm = jnp.where(m > _SCORE_THRESHOLD, m, -jnp.inf)
    sc_ref[0, 0, 0, :] = sm
    lb_ref[0, 0, 0, :] = lab


def _nms_sc_body(sc_hbm, bx_hbm, lb_hbm,
                 x1o_hbm, y1o_hbm, x2o_hbm, y2o_hbm, sco_hbm, lbo_hbm,
                 oko_hbm,
                 sc_v, bx_v, lb_v, rm_v, l2_v, x1v, y1v, x2v, y2v, scv, lbv,
                 okv, m_s, nk_s, sem):
    wid = lax.axis_index("s") * 2 + lax.axis_index("c")
    if True:
        g = wid % _B

        @pl.when(wid < _B)
        def _stage():
            pltpu.sync_copy(sc_hbm.at[g], sc_v)
            pltpu.sync_copy(bx_hbm.at[g], bx_v.at[pl.ds(0, _N * 4)])
            pltpu.sync_copy(lb_hbm.at[g], lb_v.at[pl.ds(0, _NPAD)])

        iota16 = lax.iota(jnp.int32, 16)
        padv = jnp.full((16,), _PADBOX, jnp.float32)

        for t in range(8):
            sl = pl.ds(16 * t, 16)
            x1v[sl] = padv
            y1v[sl] = padv
            x2v[sl] = padv
            y2v[sl] = padv
            scv[sl] = padv
            lbv[sl] = padv

        def rm_body(q, _):
            vec = jnp.full((16,), -jnp.inf, jnp.float32)
            for t2 in range(16):
                r = 16 * q + t2
                acc = sc_v[pl.ds(128 * r, 16)]
                for t in range(1, 8):
                    acc = jnp.maximum(acc, sc_v[pl.ds(128 * r + 16 * t, 16)])
                vec = jnp.where(iota16 == t2, _vmax(acc), vec)
            rm_v[pl.ds(16 * q, 16)] = vec
            return 0

        lax.fori_loop(0, _ROWS // 16, rm_body, 0)

        l2 = jnp.full((16,), -jnp.inf, jnp.float32)
        for q in range(10):
            l2 = jnp.where(iota16 == q, _vmax(rm_v[pl.ds(16 * q, 16)]), l2)
        l2_v[pl.ds(0, 16)] = l2
        m0 = _vmax(l2)

        m_s[0] = m0
        nk_s[0] = 0

        def inner(it2, _):
            m = m_s[0]
            nk = nk_s[0]

            @pl.when(jnp.logical_and(nk < _MAX_DET, m > -jnp.inf))
            def _step():
                l2 = l2_v[pl.ds(0, 16)]
                k3 = _vmini(jnp.where(l2 == m, iota16, _BIG))
                rchunk = rm_v[pl.ds(16 * k3, 16)]
                c = 16 * k3 + _vmini(jnp.where(rchunk == m,
                                               iota16, _BIG))
                jv = jnp.full((16,), _BIG, jnp.int32)
                for t in range(8):
                    v_t = sc_v[pl.ds(128 * c + 16 * t, 16)]
                    jv = jnp.minimum(jv, jnp.where(v_t == m,
                                                   16 * t + iota16, _BIG))
                j = _vmini(jv)
                i = 128 * c + j

                bvec = bx_v[pl.ds(4 * i, 16)]
                x1 = bvec[0]
                y1 = bvec[1]
                x2 = bvec[2]
                y2 = bvec[3]
                lab = lb_v[pl.ds(i, 16)][0]

                a1 = (x2 - x1) * (y2 - y1)
                ioumax = jnp.full((16,), -1.0, jnp.float32)
                for q in range(7):
                    sl = pl.ds(16 * q, 16)
                    x1k = x1v[sl]
                    y1k = y1v[sl]
                    x2k = x2v[sl]
                    y2k = y2v[sl]
                    ix1 = jnp.maximum(x1, x1k)
                    iy1 = jnp.maximum(y1, y1k)
                    ix2 = jnp.minimum(x2, x2k)
                    iy2 = jnp.minimum(y2, y2k)
                    inter = (jnp.maximum(ix2 - ix1, 0.0) *
                             jnp.maximum(iy2 - iy1, 0.0))
                    a2 = (x2k - x1k) * (y2k - y1k)
                    iou = inter / (a1 + a2 - inter + 1e-8)
                    ioumax = jnp.maximum(ioumax, iou)
                keep = jnp.logical_not(_vmax(ioumax) > _IOU_THRESHOLD)

                @pl.when(keep)
                def _append():
                    ksl = pl.ds(16 * (nk // 16), 16)
                    kslot = iota16 == nk % 16

                    def ins(ref, val):
                        ref[ksl] = jnp.where(kslot, val, ref[ksl])

                    ins(x1v, x1)
                    ins(y1v, y1)
                    ins(x2v, x2)
                    ins(y2v, y2)
                    ins(scv, m)
                    ins(lbv, lab)

                nk_s[0] = nk + jnp.where(keep, 1, 0)

                tstar = j // 16
                base = 128 * c + 16 * tstar
                vstar = sc_v[pl.ds(base, 16)]
                sc_v[pl.ds(base, 16)] = jnp.where(iota16 == j % 16,
                                                  -jnp.inf, vstar)
                acc = sc_v[pl.ds(128 * c, 16)]
                for t in range(1, 8):
                    acc = jnp.maximum(acc,
                                      sc_v[pl.ds(128 * c + 16 * t, 16)])
                new_rchunk = jnp.where(iota16 == c % 16, _vmax(acc), rchunk)
                rm_v[pl.ds(16 * k3, 16)] = new_rchunk
                l2n = jnp.where(iota16 == k3, _vmax(new_rchunk), l2)
                l2_v[pl.ds(0, 16)] = l2n
                m_s[0] = _vmax(l2n)

            return 0

        def outer(it, _):
            @pl.when(jnp.logical_and(nk_s[0] < _MAX_DET,
                                     m_s[0] > -jnp.inf))
            def _run():
                lax.fori_loop(0, 512, inner, 0, unroll=False)

            return 0

        lax.fori_loop(0, _NPAD // 512, outer, 0, unroll=False)
        nk = nk_s[0]

        for t in range(8):
            sl = pl.ds(16 * t, 16)
            live = (16 * t + iota16) < nk
            neg1 = jnp.full((16,), -1.0, jnp.float32)
            x1v[sl] = jnp.where(live, x1v[sl], neg1)
            y1v[sl] = jnp.where(live, y1v[sl], neg1)
            x2v[sl] = jnp.where(live, x2v[sl], neg1)
            y2v[sl] = jnp.where(live, y2v[sl], neg1)
            scv[sl] = jnp.where(live, scv[sl], neg1)
            lbv[sl] = jnp.where(live, lbv[sl], neg1)
            okv[sl] = jnp.where(live, 1.0, 0.0)

        pltpu.sync_copy(x1v, x1o_hbm.at[wid])
        pltpu.sync_copy(y1v, y1o_hbm.at[wid])
        pltpu.sync_copy(x2v, x2o_hbm.at[wid])
        pltpu.sync_copy(y2v, y2o_hbm.at[wid])
        pltpu.sync_copy(scv, sco_hbm.at[wid])
        pltpu.sync_copy(lbv, lbo_hbm.at[wid])
        pltpu.sync_copy(okv, oko_hbm.at[wid])


@jax.jit
def kernel(boxes, classification):
    nchunk = 10
    chunka = _N // nchunk
    sc4, lb4 = pl.pallas_call(
        _score_label_body,
        grid=(_B, nchunk),
        in_specs=[pl.BlockSpec((1, chunka, _NUM_CLASSES),
                               lambda b, n: (b, n, 0))],
        out_specs=[pl.BlockSpec((1, 1, 1, chunka), lambda b, n: (b, n, 0, 0)),
                   pl.BlockSpec((1, 1, 1, chunka), lambda b, n: (b, n, 0, 0))],
        out_shape=[jax.ShapeDtypeStruct((_B, nchunk, 1, chunka), jnp.float32),
                   jax.ShapeDtypeStruct((_B, nchunk, 1, chunka), jnp.float32)],
        compiler_params=pltpu.CompilerParams(
            dimension_semantics=("parallel", "parallel")),
    )(classification)

    pad = _NPAD - _N
    sc = jnp.pad(sc4.reshape(_B, _N), ((0, 0), (0, pad)),
                 constant_values=-jnp.inf).reshape(_B, _NPAD)
    lb = jnp.pad(lb4.reshape(_B, _N), ((0, 0), (0, pad))
                 ).reshape(_B, _NPAD)
    bx = boxes.reshape(_B, _N * 4)

    mesh = plsc.VectorSubcoreMesh(core_axis_name="c", subcore_axis_name="s")
    nms = pl.kernel(
        _nms_sc_body,
        out_type=[jax.ShapeDtypeStruct((32, 128), jnp.float32)] * 7,
        mesh=mesh,
        scratch_types=[
            pltpu.VMEM((_NPAD,), jnp.float32),
            pltpu.VMEM((_N * 4 + 16,), jnp.float32),
            pltpu.VMEM((_NPAD + 16,), jnp.float32),
            pltpu.VMEM((_ROWS,), jnp.float32),
            pltpu.VMEM((16,), jnp.float32),
        ] + [pltpu.VMEM((128,), jnp.float32)] * 7 + [
            pltpu.SMEM((1,), jnp.float32),
            pltpu.SMEM((1,), jnp.int32),
            pltpu.SemaphoreType.DMA,
        ],
    )
    x1o, y1o, x2o, y2o, sco, lbo, oko = [o[:_B] for o in nms(sc, bx, lb)]

    out_boxes = jnp.stack([x1o, y1o, x2o, y2o], axis=-1)[:, :_MAX_DET, :]
    out_scores = sco[:, :_MAX_DET]
    out_labels = lbo[:, :_MAX_DET].astype(jnp.int32)
    valid = jnp.sum((oko[:, :_MAX_DET] > 0.5).astype(jnp.int32), axis=1)
    return out_boxes, out_scores, out_labels, valid


# speedup vs baseline: 1.8999x; 1.8999x over previous
import functools

import jax
import jax.numpy as jnp
from jax import lax
from jax.experimental import pallas as pl
from jax.experimental.pallas import tpu as pltpu
from jax.experimental.pallas import tpu_sc as plsc

_NUM_CLASSES = 80
_SCORE_THRESHOLD = 0.05
_IOU_THRESHOLD = 0.5
_MAX_DET = 100

_N = 20000
_B = 8
_ROWS = 160
_NPAD = _ROWS * 128
_BIG = 2 ** 30
_PADBOX = 1e30


_GDN = lax.GatherDimensionNumbers(offset_dims=(), collapsed_slice_dims=(0,),
                                  start_index_map=(0,))


def _shuffle(v, idx):
    return lax.gather(v, idx[:, None], _GDN, (1,),
                      mode=lax.GatherScatterMode.PROMISE_IN_BOUNDS)


def _vmax(v):
    iota = lax.iota(jnp.int32, 16)
    for sh in (8, 4, 2, 1):
        v = jnp.maximum(v, _shuffle(v, (iota + sh) % 16))
    return v[0]


def _vmini(v):
    iota = lax.iota(jnp.int32, 16)
    for sh in (8, 4, 2, 1):
        v = jnp.minimum(v, _shuffle(v, (iota + sh) % 16))
    return v[0]

def _score_label_body(cls_ref, sc_ref, lb_ref):
    x = cls_ref[0]
    m = jnp.max(x, axis=0)
    cls_iota = jax.lax.broadcasted_iota(jnp.int32, x.shape, 0)
    lab = jnp.min(jnp.where(x == m[None, :], cls_iota, _BIG),
                  axis=0).astype(jnp.float32)
    sm = jnp.where(m > _SCORE_THRESHOLD, m, -jnp.inf)
    sc_ref[0, 0, 0, :] = jnp.concatenate(
        [sm, jnp.full((_NPAD - _N,), -jnp.inf, jnp.float32)])
    lb_ref[0, 0, 0, :] = jnp.concatenate(
        [lab, jnp.zeros((_NPAD - _N,), jnp.float32)])


def _nms_sc_body(sc_hbm, bx_hbm, lb_hbm,
                 x1o_hbm, y1o_hbm, x2o_hbm, y2o_hbm, sco_hbm, lbo_hbm,
                 oko_hbm,
                 sc_v, bx_v, lb_v, rm_v, l2_v, x1v, y1v, x2v, y2v, scv, lbv,
                 okv, m_s, nk_s, sem):
    wid = lax.axis_index("s") * 2 + lax.axis_index("c")
    if True:
        g = wid % _B

        @pl.when(wid < _B)
        def _stage():
            pltpu.sync_copy(sc_hbm.at[g], sc_v)
            pltpu.sync_copy(bx_hbm.at[g], bx_v.at[pl.ds(0, _N * 4)])
            pltpu.sync_copy(lb_hbm.at[g], lb_v.at[pl.ds(0, _NPAD)])

        iota16 = lax.iota(jnp.int32, 16)
        padv = jnp.full((16,), _PADBOX, jnp.float32)

        for t in range(8):
            sl = pl.ds(16 * t, 16)
            x1v[sl] = padv
            y1v[sl] = padv
            x2v[sl] = padv
            y2v[sl] = padv
            scv[sl] = padv
            lbv[sl] = padv

        def rm_body(q, _):
            vec = jnp.full((16,), -jnp.inf, jnp.float32)
            for t2 in range(16):
                r = 16 * q + t2
                acc = sc_v[pl.ds(128 * r, 16)]
                for t in range(1, 8):
                    acc = jnp.maximum(acc, sc_v[pl.ds(128 * r + 16 * t, 16)])
                vec = jnp.where(iota16 == t2, _vmax(acc), vec)
            rm_v[pl.ds(16 * q, 16)] = vec
            return 0

        lax.fori_loop(0, _ROWS // 16, rm_body, 0)

        l2 = jnp.full((16,), -jnp.inf, jnp.float32)
        for q in range(10):
            l2 = jnp.where(iota16 == q, _vmax(rm_v[pl.ds(16 * q, 16)]), l2)
        l2_v[pl.ds(0, 16)] = l2
        m0 = _vmax(l2)

        m_s[0] = m0
        nk_s[0] = 0

        def inner(it2, _):
            m = m_s[0]
            nk = nk_s[0]

            @pl.when(jnp.logical_and(nk < _MAX_DET, m > -jnp.inf))
            def _step():
                l2 = l2_v[pl.ds(0, 16)]
                k3 = _vmini(jnp.where(l2 == m, iota16, _BIG))
                rchunk = rm_v[pl.ds(16 * k3, 16)]
                c = 16 * k3 + _vmini(jnp.where(rchunk == m,
                                               iota16, _BIG))
                jv = jnp.full((16,), _BIG, jnp.int32)
                for t in range(8):
                    v_t = sc_v[pl.ds(128 * c + 16 * t, 16)]
                    jv = jnp.minimum(jv, jnp.where(v_t == m,
                                                   16 * t + iota16, _BIG))
                j = _vmini(jv)
                i = 128 * c + j

                bvec = bx_v[pl.ds(4 * i, 16)]
                x1 = bvec[0]
                y1 = bvec[1]
                x2 = bvec[2]
                y2 = bvec[3]
                lab = lb_v[pl.ds(i, 16)][0]

                a1 = (x2 - x1) * (y2 - y1)
                ioumax = jnp.full((16,), -1.0, jnp.float32)
                for q in range(7):
                    sl = pl.ds(16 * q, 16)
                    x1k = x1v[sl]
                    y1k = y1v[sl]
                    x2k = x2v[sl]
                    y2k = y2v[sl]
                    ix1 = jnp.maximum(x1, x1k)
                    iy1 = jnp.maximum(y1, y1k)
                    ix2 = jnp.minimum(x2, x2k)
                    iy2 = jnp.minimum(y2, y2k)
                    inter = (jnp.maximum(ix2 - ix1, 0.0) *
                             jnp.maximum(iy2 - iy1, 0.0))
                    a2 = (x2k - x1k) * (y2k - y1k)
                    iou = inter / (a1 + a2 - inter + 1e-8)
                    ioumax = jnp.maximum(ioumax, iou)
                keep = jnp.logical_not(_vmax(ioumax) > _IOU_THRESHOLD)

                @pl.when(keep)
                def _append():
                    ksl = pl.ds(16 * (nk // 16), 16)
                    kslot = iota16 == nk % 16

                    def ins(ref, val):
                        ref[ksl] = jnp.where(kslot, val, ref[ksl])

                    ins(x1v, x1)
                    ins(y1v, y1)
                    ins(x2v, x2)
                    ins(y2v, y2)
                    ins(scv, m)
                    ins(lbv, lab)

                nk_s[0] = nk + jnp.where(keep, 1, 0)

                tstar = j // 16
                base = 128 * c + 16 * tstar
                vstar = sc_v[pl.ds(base, 16)]
                sc_v[pl.ds(base, 16)] = jnp.where(iota16 == j % 16,
                                                  -jnp.inf, vstar)
                acc = sc_v[pl.ds(128 * c, 16)]
                for t in range(1, 8):
                    acc = jnp.maximum(acc,
                                      sc_v[pl.ds(128 * c + 16 * t, 16)])
                new_rchunk = jnp.where(iota16 == c % 16, _vmax(acc), rchunk)
                rm_v[pl.ds(16 * k3, 16)] = new_rchunk
                l2n = jnp.where(iota16 == k3, _vmax(new_rchunk), l2)
                l2_v[pl.ds(0, 16)] = l2n
                m_s[0] = _vmax(l2n)

            return 0

        def outer(it, _):
            @pl.when(jnp.logical_and(nk_s[0] < _MAX_DET,
                                     m_s[0] > -jnp.inf))
            def _run():
                lax.fori_loop(0, 512, inner, 0, unroll=False)

            return 0

        lax.fori_loop(0, _NPAD // 512, outer, 0, unroll=False)
        nk = nk_s[0]

        for t in range(8):
            sl = pl.ds(16 * t, 16)
            live = (16 * t + iota16) < nk
            neg1 = jnp.full((16,), -1.0, jnp.float32)
            x1v[sl] = jnp.where(live, x1v[sl], neg1)
            y1v[sl] = jnp.where(live, y1v[sl], neg1)
            x2v[sl] = jnp.where(live, x2v[sl], neg1)
            y2v[sl] = jnp.where(live, y2v[sl], neg1)
            scv[sl] = jnp.where(live, scv[sl], neg1)
            lbv[sl] = jnp.where(live, lbv[sl], neg1)
            okv[sl] = jnp.where(live, 1.0, 0.0)

        pltpu.sync_copy(x1v, x1o_hbm.at[wid])
        pltpu.sync_copy(y1v, y1o_hbm.at[wid])
        pltpu.sync_copy(x2v, x2o_hbm.at[wid])
        pltpu.sync_copy(y2v, y2o_hbm.at[wid])
        pltpu.sync_copy(scv, sco_hbm.at[wid])
        pltpu.sync_copy(lbv, lbo_hbm.at[wid])
        pltpu.sync_copy(okv, oko_hbm.at[wid])


@jax.jit
def kernel(boxes, classification):
    cls_t = jnp.transpose(classification, (0, 2, 1))
    sc4, lb4 = pl.pallas_call(
        _score_label_body,
        grid=(_B,),
        in_specs=[pl.BlockSpec((1, _NUM_CLASSES, _N), lambda b: (b, 0, 0))],
        out_specs=[pl.BlockSpec((1, 1, 1, _NPAD), lambda b: (b, 0, 0, 0)),
                   pl.BlockSpec((1, 1, 1, _NPAD), lambda b: (b, 0, 0, 0))],
        out_shape=[jax.ShapeDtypeStruct((_B, 1, 1, _NPAD), jnp.float32),
                   jax.ShapeDtypeStruct((_B, 1, 1, _NPAD), jnp.float32)],
        compiler_params=pltpu.CompilerParams(
            dimension_semantics=("parallel",)),
    )(cls_t)

    sc = sc4.reshape(_B, _NPAD)
    lb = lb4.reshape(_B, _NPAD)
    bx = boxes.reshape(_B, _N * 4)

    mesh = plsc.VectorSubcoreMesh(core_axis_name="c", subcore_axis_name="s")
    nms = pl.kernel(
        _nms_sc_body,
        out_type=[jax.ShapeDtypeStruct((32, 128), jnp.float32)] * 7,
        mesh=mesh,
        scratch_types=[
            pltpu.VMEM((_NPAD,), jnp.float32),
            pltpu.VMEM((_N * 4 + 16,), jnp.float32),
            pltpu.VMEM((_NPAD + 16,), jnp.float32),
            pltpu.VMEM((_ROWS,), jnp.float32),
            pltpu.VMEM((16,), jnp.float32),
        ] + [pltpu.VMEM((128,), jnp.float32)] * 7 + [
            pltpu.SMEM((1,), jnp.float32),
            pltpu.SMEM((1,), jnp.int32),
            pltpu.SemaphoreType.DMA,
        ],
    )
    x1o, y1o, x2o, y2o, sco, lbo, oko = [o[:_B] for o in nms(sc, bx, lb)]

    out_boxes = jnp.stack([x1o, y1o, x2o, y2o], axis=-1)[:, :_MAX_DET, :]
    out_scores = sco[:, :_MAX_DET]
    out_labels = lbo[:, :_MAX_DET].astype(jnp.int32)
    valid = jnp.sum((oko[:, :_MAX_DET] > 0.5).astype(jnp.int32), axis=1)
    return out_boxes, out_scores, out_labels, valid
